# Initial kernel scaffold; baseline (speedup 1.0000x reference)
import jax
import jax.numpy as jnp
from jax.experimental import pallas as pl


def kernel(x_chaos, W_gat, att_src, att_dst, b_gat, gamma1, beta1, W_res, b_res, eps_gin, W_g1, b_g1, gamma2, beta2, W_g2, b_g2, W_fin, b_fin):
    raise NotImplementedError("write your pallas kernel here")


# speedup vs baseline: 43.0726x; 43.0726x over previous
import functools

import jax
import jax.numpy as jnp
from jax import lax
from jax.experimental import pallas as pl
from jax.experimental.pallas import tpu as pltpu
from jax.experimental.pallas import tpu_sc as plsc

_B, _L, _M, _D = 4, 64, 40, 16
_H, _C, _O = 4, 128, 256
_K = 3
_PRED, _ENC = 96, 7
_NPER = _L * _M
_N = _B * _NPER

_NC, _NS = 2, 16
_NW = _NC * _NS

_RK = 256
_RG = 512


def _leaky(e):
    return jnp.where(e > 0, e, 0.2 * e)


def _knn_body(xr_ref, xa_ref, o_ref):
    b = pl.program_id(0)
    i = pl.program_id(1)
    xr = xr_ref[0]
    xa = xa_ref[0]
    xr_aug = jnp.concatenate([xr, jnp.ones((_RK, 1), jnp.float32)], axis=1)
    sqa = jnp.sum(xa * xa, axis=1, keepdims=True)
    xa_aug = jnp.concatenate([-2.0 * xa, sqa], axis=1)
    d2 = lax.dot_general(xr_aug, xa_aug, (((1,), (1,)), ((), ())),
                         preferred_element_type=jnp.float32)
    col = lax.broadcasted_iota(jnp.int32, (_RK, _NPER), 1)
    row = i * _RK + lax.broadcasted_iota(jnp.int32, (_RK, _NPER), 0)
    d2 = d2 + jnp.where(col == row, 1e9, 0.0)
    sels = []
    for _ in range(_K):
        m = jnp.min(d2, axis=1, keepdims=True)
        sel = jnp.min(jnp.where(d2 == m, col, jnp.int32(2**30)),
                      axis=1, keepdims=True)
        sels.append(sel)
        d2 = jnp.where(col == sel, jnp.float32(jnp.inf), d2)
    base = b * _NPER
    pad = jnp.zeros((_RK, 8 - _K), jnp.int32)
    o_ref[...] = jnp.concatenate([s + base for s in sels] + [pad], axis=1)


def _knn_call(x3):
    nt = _NPER // _RK
    return pl.pallas_call(
        _knn_body,
        grid=(_B, nt),
        in_specs=[
            pl.BlockSpec((1, _RK, _D), lambda b, i: (b, i, 0)),
            pl.BlockSpec((1, _NPER, _D), lambda b, i: (b, 0, 0)),
        ],
        out_specs=pl.BlockSpec((_RK, 8), lambda b, i: (b * nt + i, 0)),
        out_shape=jax.ShapeDtypeStruct((_N, 8), jnp.int32),
    )(x3, x3)


def _sc_gather_rows(xpad, idx3):
    per = _N // _NW
    ch = 64
    nch = per // ch
    mesh = plsc.VectorSubcoreMesh(core_axis_name="c", subcore_axis_name="s")

    @functools.partial(
        pl.kernel, mesh=mesh,
        out_type=jax.ShapeDtypeStruct((_K, _N, _D), jnp.float32),
        scratch_types=[
            pltpu.VMEM((ch,), jnp.int32),
            pltpu.VMEM((ch, _C), jnp.float32),
            pltpu.VMEM((ch, _D), jnp.float32),
            pltpu.SemaphoreType.DMA,
        ],
    )
    def k(x_hbm, idx_hbm, out_hbm, idx_v, rows_v, cmp_v, sem):
        wid = lax.axis_index("s") * _NC + lax.axis_index("c")
        base = wid * per
        for kk in range(_K):
            for j in range(nch):
                off = base + j * ch
                pltpu.sync_copy(idx_hbm.at[kk, pl.ds(off, ch)], idx_v)
                pltpu.async_copy(x_hbm.at[idx_v], rows_v, sem).wait()
                for i in range(ch):
                    cmp_v[i, pl.ds(0, _D)] = rows_v[i, pl.ds(0, _D)]
                pltpu.sync_copy(cmp_v, out_hbm.at[kk, pl.ds(off, ch), :])

    return k(xpad, idx3)


def _gat_body(x_ref, xg_ref, wg_ref, a_ref, e_ref, bg_ref, o_ref):
    wg = wg_ref[...]
    amat = a_ref[...]
    emat = e_ref[...]
    xp_s = jnp.dot(x_ref[...], wg, preferred_element_type=jnp.float32)
    a_self = jnp.dot(xp_s, amat, preferred_element_type=jnp.float32)
    ad = a_self[:, 4:8]
    elist = [_leaky(a_self[:, 0:4] + ad)]
    xps = [xp_s]
    for kk in range(_K):
        xp_k = jnp.dot(xg_ref[kk], wg, preferred_element_type=jnp.float32)
        a_k = jnp.dot(xp_k, amat, preferred_element_type=jnp.float32)
        elist.append(_leaky(a_k[:, 0:4] + ad))
        xps.append(xp_k)
    m = jnp.maximum(jnp.maximum(elist[0], elist[1]),
                    jnp.maximum(elist[2], elist[3]))
    exs = [jnp.exp(e - m) for e in elist]
    ssum = exs[0] + exs[1] + exs[2] + exs[3]
    acc = None
    for ex, xp in zip(exs, xps):
        w = jnp.dot(ex / ssum, emat, preferred_element_type=jnp.float32)
        acc = w * xp if acc is None else acc + w * xp
    outm = (acc[:, 0:_C] + acc[:, _C:2 * _C]
            + acc[:, 2 * _C:3 * _C] + acc[:, 3 * _C:4 * _C]) * 0.25
    o_ref[...] = outm + bg_ref[...]


def _gat_call(x, xg, w_gat, amat, emat, bg2d):
    nt = _N // _RG
    return pl.pallas_call(
        _gat_body,
        grid=(nt,),
        in_specs=[
            pl.BlockSpec((_RG, _D), lambda i: (i, 0)),
            pl.BlockSpec((_K, _RG, _D), lambda i: (0, i, 0)),
            pl.BlockSpec((_D, _H * _C), lambda i: (0, 0)),
            pl.BlockSpec((_H * _C, 8), lambda i: (0, 0)),
            pl.BlockSpec((_H, _H * _C), lambda i: (0, 0)),
            pl.BlockSpec((1, _C), lambda i: (0, 0)),
        ],
        out_specs=pl.BlockSpec((_RG, _C), lambda i: (i, 0)),
        out_shape=jax.ShapeDtypeStruct((_N, _C), jnp.float32),
    )(x, xg, w_gat, amat, emat, bg2d)


def _bn_body(x_ref, g_ref, b_ref, o_ref):
    x = x_ref[...]
    mu = jnp.mean(x, axis=0, keepdims=True)
    xc = x - mu
    var = jnp.mean(xc * xc, axis=0, keepdims=True)
    y = xc * lax.rsqrt(var + 1e-5) * g_ref[...] + b_ref[...]
    o_ref[...] = jnp.maximum(y, 0.0)


def _bn_call(xpre, g2d, b2d):
    return pl.pallas_call(
        _bn_body,
        out_shape=jax.ShapeDtypeStruct((_N, _C), jnp.float32),
    )(xpre, g2d, b2d)


def _sc_agg(x_gat, idx3):
    per = _N // _NW
    ch = 32
    nch = per // ch
    nv = _C // 16
    mesh = plsc.VectorSubcoreMesh(core_axis_name="c", subcore_axis_name="s")

    @functools.partial(
        pl.kernel, mesh=mesh,
        out_type=jax.ShapeDtypeStruct((_N, _C), jnp.float32),
        scratch_types=[
            pltpu.VMEM((ch,), jnp.int32),
            pltpu.VMEM((ch, _C), jnp.float32),
            pltpu.VMEM((ch, _C), jnp.float32),
            pltpu.VMEM((ch, _C), jnp.float32),
            pltpu.VMEM((ch, _C), jnp.float32),
            pltpu.SemaphoreType.DMA,
        ],
    )
    def k(xg_hbm, idx_hbm, out_hbm, idx_v, r0, r1, r2, acc, sem):
        wid = lax.axis_index("s") * _NC + lax.axis_index("c")
        base = wid * per
        rbufs = [r0, r1, r2]

        def body(j, carry):
            off = base + j * ch
            for kk in range(_K):
                pltpu.sync_copy(idx_hbm.at[kk, pl.ds(off, ch)], idx_v)
                pltpu.async_copy(xg_hbm.at[idx_v], rbufs[kk], sem).wait()
            for i in range(ch):
                for v in range(nv):
                    sl = pl.ds(v * 16, 16)
                    acc[i, sl] = r0[i, sl] + r1[i, sl] + r2[i, sl]
            pltpu.sync_copy(acc, out_hbm.at[pl.ds(off, ch), :])
            return carry

        lax.fori_loop(0, nch, body, 0)

    return k(x_gat, idx3)


def _final_body(xg_ref, agg_ref, eps_ref, wg1_ref, bg1_ref, g2_ref, be2_ref,
                wg2_ref, bg2_ref, wr_ref, br_ref, wf_ref, bf_ref, o_ref):
    xgat = xg_ref[...]
    h = xgat * (1.0 + eps_ref[...]) + agg_ref[...]
    t = jnp.dot(h, wg1_ref[...], preferred_element_type=jnp.float32) + bg1_ref[...]
    mu = jnp.mean(t, axis=0, keepdims=True)
    tc = t - mu
    var = jnp.mean(tc * tc, axis=0, keepdims=True)
    t = jnp.maximum(tc * lax.rsqrt(var + 1e-5) * g2_ref[...] + be2_ref[...], 0.0)
    x_gin = jnp.dot(t, wg2_ref[...], preferred_element_type=jnp.float32) + bg2_ref[...]
    x_res = jnp.dot(xgat, wr_ref[...], preferred_element_type=jnp.float32) + br_ref[...]
    node = jnp.maximum(x_gin + x_res, 0.0)
    gs = [jnp.mean(node[bb * _NPER:(bb + 1) * _NPER, :], axis=0, keepdims=True)
          for bb in range(_B)]
    g = jnp.concatenate(gs, axis=0)
    o_ref[...] = jnp.dot(g, wf_ref[...], preferred_element_type=jnp.float32) + bf_ref[...]


def _final_call(x_gat, agg, eps2d, w_g1, bg1, g2, be2, w_g2, bg2, w_res, br,
                w_fin, bf):
    return pl.pallas_call(
        _final_body,
        out_shape=jax.ShapeDtypeStruct((_B, _PRED * _ENC), jnp.float32),
    )(x_gat, agg, eps2d, w_g1, bg1, g2, be2, w_g2, bg2, w_res, br, w_fin, bf)


def kernel(x_chaos, W_gat, att_src, att_dst, b_gat, gamma1, beta1, W_res,
           b_res, eps_gin, W_g1, b_g1, gamma2, beta2, W_g2, b_g2, W_fin,
           b_fin):
    x = x_chaos.reshape(_N, _D).astype(jnp.float32)
    x3 = x.reshape(_B, _NPER, _D)

    idx8 = _knn_call(x3)
    idx3 = idx8[:, :_K].T

    xpad = jnp.pad(x, ((0, 0), (0, _C - _D)))
    xg = _sc_gather_rows(xpad, idx3)

    eye = jnp.eye(_H, dtype=jnp.float32)
    amat = jnp.concatenate(
        [(att_src[:, :, None] * eye[:, None, :]).reshape(_H * _C, _H),
         (att_dst[:, :, None] * eye[:, None, :]).reshape(_H * _C, _H)], axis=1)
    emat = jnp.repeat(eye, _C, axis=1)

    out_pre = _gat_call(x, xg, W_gat, amat, emat, b_gat.reshape(1, _C))
    x_gat = _bn_call(out_pre, gamma1.reshape(1, _C), beta1.reshape(1, _C))

    agg = _sc_agg(x_gat, idx3)

    proj = _final_call(
        x_gat, agg, eps_gin.reshape(1, 1), W_g1, b_g1.reshape(1, _C),
        gamma2.reshape(1, _C), beta2.reshape(1, _C), W_g2,
        b_g2.reshape(1, _O), W_res, b_res.reshape(1, _O), W_fin,
        b_fin.reshape(1, _PRED * _ENC))
    return proj.reshape(_B, _PRED, _ENC)
